# baseline (speedup 1.0000x reference)
import jax
import jax.numpy as jnp
from jax.experimental import pallas as pl
from jax.experimental.pallas import tpu as pltpu

_EPS = 1e-5
_MB = 256
_BIG = 3.0e38


def _dot(a, b):
    return jnp.dot(a, b, preferred_element_type=jnp.float32)


def _acc(ref, val, first):
    @pl.when(first)
    def _():
        ref[...] = jnp.zeros_like(ref)

    ref[...] += val


def _stats(hs, ms, s_ref, q_ref, first):
    hmacc = hs[0] * ms[0]
    qacc = hmacc * hs[0]
    for n in range(1, len(hs)):
        hm = hs[n] * ms[n]
        hmacc = hmacc + hm
        qacc = qacc + hm * hs[n]
    ones = jnp.ones((hmacc.shape[1], 1), jnp.float32)
    _acc(s_ref, _dot(hmacc, ones), first)
    _acc(q_ref, _dot(qacc, ones), first)


def _finalize(s_ref, q_ref, cnt, g_ref, b_ref, sc_ref, sh_ref):
    mean = s_ref[...] / cnt
    var = jnp.maximum(q_ref[...] / cnt - mean * mean, 0.0)
    sc = g_ref[...] * jax.lax.rsqrt(var + _EPS)
    sc_ref[...] = sc
    sh_ref[...] = b_ref[...] - mean * sc


def _feats(x_ref, m, wp, sc0, sh0):
    out = []
    for n in range(x_ref.shape[0]):
        h1 = _dot(wp, x_ref[n])
        out.append(jnp.maximum(h1 * sc0 + sh0, 0.0) * m[n])
    return out


def _body(x_ref, m_ref, wp_ref, g0_ref, b0_ref, w1a_ref, w1b_ref,
          g1_ref, b1_ref, w2_ref, g2_ref, b2_ref, wo1_ref, bo1_ref,
          waug_ref, o_ref,
          s0, q0, c0, s1, q1, s2, q2, sc0, sh0, sc1, sh1, sc2, sh2,
          pb_all, bm_all):
    p = pl.program_id(0)
    i = pl.program_id(1)
    first = i == 0
    n_pts = x_ref.shape[0]
    blk = pl.ds(i * _MB, _MB)

    @pl.when(p == 0)
    def _phase0():
        wp = wp_ref[...]
        m = m_ref[...]
        hs = [_dot(wp, x_ref[n]) for n in range(n_pts)]
        _stats(hs, [m[n] for n in range(n_pts)], s0, q0, first)
        _acc(c0, jnp.sum(m).reshape(1, 1), first)

    @pl.when(p == 1)
    def _phase1():
        @pl.when(first)
        def _():
            _finalize(s0, q0, jnp.maximum(c0[0, 0], 1.0),
                      g0_ref, b0_ref, sc0, sh0)

        m = m_ref[...]
        feats = _feats(x_ref, m, wp_ref[...], sc0[...], sh0[...])
        pooled = feats[0]
        for n in range(1, n_pts):
            pooled = jnp.maximum(pooled, feats[n])
        pb = _dot(w1b_ref[...], pooled)
        pb_all[:, blk] = pb
        w1a = w1a_ref[...]
        h2s = [_dot(w1a, feats[n]) + pb for n in range(n_pts)]
        _stats(h2s, [m[n] for n in range(n_pts)], s1, q1, first)

    @pl.when(p == 2)
    def _phase2():
        @pl.when(first)
        def _():
            _finalize(s1, q1, jnp.maximum(c0[0, 0], 1.0),
                      g1_ref, b1_ref, sc1, sh1)

        m = m_ref[...]
        feats = _feats(x_ref, m, wp_ref[...], sc0[...], sh0[...])
        pb = pb_all[:, blk]
        w1a = w1a_ref[...]
        w2 = w2_ref[...]
        _sc1 = sc1[...]
        _sh1 = sh1[...]
        h3s = []
        bmacc = None
        for n in range(n_pts):
            h2 = _dot(w1a, feats[n]) + pb
            t2 = jnp.maximum(h2 * _sc1 + _sh1, 0.0)
            h3 = _dot(w2, t2)
            h3s.append(h3)
            h3m = jnp.where(m[n] != 0.0, h3, -_BIG)
            bmacc = h3m if bmacc is None else jnp.maximum(bmacc, h3m)
        bm_all[:, blk] = bmacc
        _stats(h3s, [m[n] for n in range(n_pts)], s2, q2, first)

    @pl.when(p == 3)
    def _phase3():
        @pl.when(first)
        def _():
            _finalize(s2, q2, jnp.maximum(c0[0, 0], 1.0),
                      g2_ref, b2_ref, sc2, sh2)

        bm = bm_all[:, blk]
        buf = jnp.maximum(bm * sc2[...] + sh2[...], 0.0)
        t = jnp.maximum(_dot(wo1_ref[...], buf) + bo1_ref[...], 0.0)
        validf = (bm[0:1, :] > -1e37).astype(jnp.float32)
        ta = jnp.concatenate([t * validf, validf], axis=0)
        o_ref[...] = _dot(ta.T, waug_ref[...])


def kernel(polylines, polylines_mask, W_pre, g_pre, b_pre, W1, g1, b1,
           W2, g2, b2, Wo1, bo1, Wo2, bo2):
    B, P, N, C = polylines.shape
    H = W_pre.shape[0]
    O = Wo2.shape[0]
    M = B * P
    f32 = jnp.float32

    xt = polylines.reshape(M, N, C).transpose(1, 2, 0)
    mt = polylines_mask.reshape(M, N).T.reshape(N, 1, M).astype(f32)
    waug = jnp.concatenate([Wo2.T, bo2[None, :]], axis=0)

    G = M // _MB
    grid = (4, G)

    def full(shp):
        return pl.BlockSpec(shp, lambda p, i: tuple(0 for _ in shp))

    x_spec = pl.BlockSpec(
        (N, C, _MB), lambda p, i: (0, 0, jnp.where(p <= 2, i, 0)))
    m_spec = pl.BlockSpec(
        (N, 1, _MB), lambda p, i: (0, 0, jnp.where(p <= 2, i, 0)))
    o_spec = pl.BlockSpec(
        (_MB, O), lambda p, i: (jnp.where(p == 3, i, 0), 0))

    colH = lambda: pltpu.VMEM((H, 1), f32)
    out = pl.pallas_call(
        _body,
        grid=grid,
        in_specs=[x_spec, m_spec, full((H, C)), full((H, 1)), full((H, 1)),
                  full((H, H)), full((H, H)), full((H, 1)), full((H, 1)),
                  full((H, H)), full((H, 1)), full((H, 1)),
                  full((H, H)), full((H, 1)), full((H + 1, O))],
        out_specs=o_spec,
        out_shape=jax.ShapeDtypeStruct((M, O), f32),
        scratch_shapes=[
            colH(), colH(), pltpu.VMEM((1, 1), f32),
            colH(), colH(), colH(), colH(),
            colH(), colH(), colH(), colH(), colH(), colH(),
            pltpu.VMEM((H, M), f32),
            pltpu.VMEM((H, M), f32),
        ],
    )(xt, mt, W_pre, g_pre.reshape(H, 1), b_pre.reshape(H, 1),
      W1[:, :H], W1[:, H:], g1.reshape(H, 1), b1.reshape(H, 1),
      W2, g2.reshape(H, 1), b2.reshape(H, 1),
      Wo1, bo1.reshape(H, 1), waug)

    return out.reshape(B, P, O)


# speedup vs baseline: 1.0663x; 1.0663x over previous
import jax
import jax.numpy as jnp
from jax.experimental import pallas as pl
from jax.experimental.pallas import tpu as pltpu

_EPS = 1e-5
_MB = 256
_BIG = 3.0e38


def _dot(a, b):
    return jnp.dot(a, b, preferred_element_type=jnp.float32)


def _acc(ref, val, first):
    @pl.when(first)
    def _():
        ref[...] = jnp.zeros_like(ref)

    ref[...] += val


def _stats(hs, ms, s_ref, q_ref, first):
    hmacc = hs[0] * ms[0]
    qacc = hmacc * hs[0]
    for n in range(1, len(hs)):
        hm = hs[n] * ms[n]
        hmacc = hmacc + hm
        qacc = qacc + hm * hs[n]
    ones = jnp.ones((hmacc.shape[1], 1), jnp.float32)
    _acc(s_ref, _dot(hmacc, ones), first)
    _acc(q_ref, _dot(qacc, ones), first)


def _finalize(s_ref, q_ref, cnt, g_ref, b_ref, sc_ref, sh_ref):
    mean = s_ref[...] / cnt
    var = jnp.maximum(q_ref[...] / cnt - mean * mean, 0.0)
    sc = g_ref[...] * jax.lax.rsqrt(var + _EPS)
    sc_ref[...] = sc
    sh_ref[...] = b_ref[...] - mean * sc


def _feats(x_ref, m, wp, sc0, sh0):
    out = []
    for n in range(x_ref.shape[0]):
        h1 = _dot(wp, x_ref[n])
        out.append(jnp.maximum(h1 * sc0 + sh0, 0.0) * m[n])
    return out


def _body(x_ref, m_ref, wp_ref, g0_ref, b0_ref, w1a_ref, w1b_ref,
          g1_ref, b1_ref, w2_ref, g2_ref, b2_ref, wo1_ref, bo1_ref,
          waug_ref, o_ref,
          s0, q0, c0, s1, q1, s2, q2, sc0, sh0, sc1, sh1, sc2, sh2,
          h2_all, bm_all):
    p = pl.program_id(0)
    i = pl.program_id(1)
    first = i == 0
    n_pts = x_ref.shape[0]
    blk = pl.ds(i * _MB, _MB)

    @pl.when(p == 0)
    def _phase0():
        wp = wp_ref[...]
        m = m_ref[...]
        hs = [_dot(wp, x_ref[n]) for n in range(n_pts)]
        _stats(hs, [m[n] for n in range(n_pts)], s0, q0, first)
        _acc(c0, jnp.sum(m).reshape(1, 1), first)

    @pl.when(p == 1)
    def _phase1():
        @pl.when(first)
        def _():
            _finalize(s0, q0, jnp.maximum(c0[0, 0], 1.0),
                      g0_ref, b0_ref, sc0, sh0)

        m = m_ref[...]
        feats = _feats(x_ref, m, wp_ref[...], sc0[...], sh0[...])
        pooled = feats[0]
        for n in range(1, n_pts):
            pooled = jnp.maximum(pooled, feats[n])
        pb = _dot(w1b_ref[...], pooled)
        w1a = w1a_ref[...]
        h2s = []
        for n in range(n_pts):
            h2 = _dot(w1a, feats[n]) + pb
            h2_all[n, :, blk] = h2.astype(jnp.bfloat16)
            h2s.append(h2)
        _stats(h2s, [m[n] for n in range(n_pts)], s1, q1, first)

    @pl.when(p == 2)
    def _phase2():
        @pl.when(first)
        def _():
            _finalize(s1, q1, jnp.maximum(c0[0, 0], 1.0),
                      g1_ref, b1_ref, sc1, sh1)

        m = m_ref[...]
        w2 = w2_ref[...]
        _sc1 = sc1[...]
        _sh1 = sh1[...]
        h3s = []
        bmacc = None
        for n in range(n_pts):
            h2 = h2_all[n, :, blk].astype(jnp.float32)
            t2 = jnp.maximum(h2 * _sc1 + _sh1, 0.0)
            h3 = _dot(w2, t2)
            h3s.append(h3)
            h3m = jnp.where(m[n] != 0.0, h3, -_BIG)
            bmacc = h3m if bmacc is None else jnp.maximum(bmacc, h3m)
        bm_all[:, blk] = bmacc
        _stats(h3s, [m[n] for n in range(n_pts)], s2, q2, first)

    @pl.when(p == 3)
    def _phase3():
        @pl.when(first)
        def _():
            _finalize(s2, q2, jnp.maximum(c0[0, 0], 1.0),
                      g2_ref, b2_ref, sc2, sh2)

        bm = bm_all[:, blk]
        buf = jnp.maximum(bm * sc2[...] + sh2[...], 0.0)
        t = jnp.maximum(_dot(wo1_ref[...], buf) + bo1_ref[...], 0.0)
        validf = (bm[0:1, :] > -1e37).astype(jnp.float32)
        ta = jnp.concatenate([t * validf, validf], axis=0)
        o_ref[...] = _dot(ta.T, waug_ref[...])


def kernel(polylines, polylines_mask, W_pre, g_pre, b_pre, W1, g1, b1,
           W2, g2, b2, Wo1, bo1, Wo2, bo2):
    B, P, N, C = polylines.shape
    H = W_pre.shape[0]
    O = Wo2.shape[0]
    M = B * P
    f32 = jnp.float32

    xt = polylines.reshape(M, N, C).transpose(1, 2, 0)
    mt = polylines_mask.reshape(M, N).T.reshape(N, 1, M).astype(f32)
    waug = jnp.concatenate([Wo2.T, bo2[None, :]], axis=0)

    G = M // _MB
    grid = (4, G)

    def full(shp):
        return pl.BlockSpec(shp, lambda p, i: tuple(0 for _ in shp))

    x_spec = pl.BlockSpec(
        (N, C, _MB), lambda p, i: (0, 0, jnp.where(p <= 1, i, 0)))
    m_spec = pl.BlockSpec(
        (N, 1, _MB), lambda p, i: (0, 0, jnp.where(p <= 2, i, 0)))
    o_spec = pl.BlockSpec(
        (_MB, O), lambda p, i: (jnp.where(p == 3, i, 0), 0))

    colH = lambda: pltpu.VMEM((H, 1), f32)
    out = pl.pallas_call(
        _body,
        grid=grid,
        in_specs=[x_spec, m_spec, full((H, C)), full((H, 1)), full((H, 1)),
                  full((H, H)), full((H, H)), full((H, 1)), full((H, 1)),
                  full((H, H)), full((H, 1)), full((H, 1)),
                  full((H, H)), full((H, 1)), full((H + 1, O))],
        out_specs=o_spec,
        out_shape=jax.ShapeDtypeStruct((M, O), f32),
        scratch_shapes=[
            colH(), colH(), pltpu.VMEM((1, 1), f32),
            colH(), colH(), colH(), colH(),
            colH(), colH(), colH(), colH(), colH(), colH(),
            pltpu.VMEM((N, H, M), jnp.bfloat16),
            pltpu.VMEM((H, M), f32),
        ],
    )(xt, mt, W_pre, g_pre.reshape(H, 1), b_pre.reshape(H, 1),
      W1[:, :H], W1[:, H:], g1.reshape(H, 1), b1.reshape(H, 1),
      W2, g2.reshape(H, 1), b2.reshape(H, 1),
      Wo1, bo1.reshape(H, 1), waug)

    return out.reshape(B, P, O)
